# Initial kernel scaffold; baseline (speedup 1.0000x reference)
import functools

import jax
import jax.numpy as jnp
from jax import lax
from jax.experimental import pallas as pl
from jax.experimental.pallas import tpu as pltpu
from jax.experimental.pallas import tpu_sc as plsc

B, N_NODES, DIM, MAX_Z = 256, 64, 128, 100
NF = B * N_NODES
E = B * N_NODES * N_NODES

NW = 32
E_PER_W = E // NW
CHUNK = 8192
N_CHUNKS = E_PER_W // CHUNK
LANES = 16

_sc_mesh = plsc.VectorSubcoreMesh(core_axis_name="c", subcore_axis_name="s")


@functools.partial(
    pl.kernel,
    mesh=_sc_mesh,
    out_type=jax.ShapeDtypeStruct((E,), jnp.float32),
    scratch_types=[
        pltpu.VMEM((NF,), jnp.float32),
        pltpu.VMEM((NF,), jnp.float32),
        pltpu.VMEM((NF,), jnp.float32),
        pltpu.VMEM((CHUNK,), jnp.int32),
        pltpu.VMEM((CHUNK,), jnp.int32),
        pltpu.VMEM((CHUNK,), jnp.float32),
    ],
)
def _dist_sc(x0_hbm, x1_hbm, x2_hbm, row_hbm, col_hbm, out_hbm,
             x0_v, x1_v, x2_v, row_v, col_v, out_v):
    wid = lax.axis_index("s") * 2 + lax.axis_index("c")
    base = wid * E_PER_W
    pltpu.sync_copy(x0_hbm, x0_v)
    pltpu.sync_copy(x1_hbm, x1_v)
    pltpu.sync_copy(x2_hbm, x2_v)

    def chunk_body(ci, carry):
        off = base + ci * CHUNK
        pltpu.sync_copy(row_hbm.at[pl.ds(off, CHUNK)], row_v)
        pltpu.sync_copy(col_hbm.at[pl.ds(off, CHUNK)], col_v)

        def step(i, c2):
            r = row_v[pl.ds(i * LANES, LANES)]
            c = col_v[pl.ds(i * LANES, LANES)]
            dx = plsc.load_gather(x0_v, [r]) - plsc.load_gather(x0_v, [c])
            dy = plsc.load_gather(x1_v, [r]) - plsc.load_gather(x1_v, [c])
            dz = plsc.load_gather(x2_v, [r]) - plsc.load_gather(x2_v, [c])
            out_v[pl.ds(i * LANES, LANES)] = dx * dx + dy * dy + dz * dz
            return c2

        lax.fori_loop(0, CHUNK // LANES, step, 0)
        pltpu.sync_copy(out_v, out_hbm.at[pl.ds(off, CHUNK)])
        return carry

    lax.fori_loop(0, N_CHUNKS, chunk_body, 0)


ROWS = 1024
GRID = NF // ROWS


def _param_body(cat_ref, ch_ref, nm_ref, emb_ref, w_ref, b_ref, out_ref):
    cat = cat_ref[...]
    z = lax.broadcasted_iota(jnp.float32, (1, DIM), 1)
    onehot = (cat == z).astype(jnp.float32)
    h = jnp.dot(onehot, emb_ref[...], preferred_element_type=jnp.float32)
    lane = lax.broadcasted_iota(jnp.int32, (ROWS, DIM), 1)
    h = jnp.where(lane == 0, ch_ref[...], h)
    h = h * nm_ref[...]
    out_ref[...] = (
        jnp.dot(h, w_ref[...], preferred_element_type=jnp.float32) + b_ref[...]
    )


_param_tc = pl.pallas_call(
    _param_body,
    grid=(GRID,),
    in_specs=[
        pl.BlockSpec((ROWS, 1), lambda i: (i, 0)),
        pl.BlockSpec((ROWS, 1), lambda i: (i, 0)),
        pl.BlockSpec((ROWS, 1), lambda i: (i, 0)),
        pl.BlockSpec((DIM, DIM), lambda i: (0, 0)),
        pl.BlockSpec((DIM, 2 * DIM), lambda i: (0, 0)),
        pl.BlockSpec((1, 2 * DIM), lambda i: (0, 0)),
    ],
    out_specs=pl.BlockSpec((ROWS, 2 * DIM), lambda i: (i, 0)),
    out_shape=jax.ShapeDtypeStruct((NF, 2 * DIM), jnp.float32),
)


def kernel(x, categories, charges, edges, node_mask, edge_mask,
           emb_table, W_ml, b_ml):
    xf = x.reshape(NF, 3)
    x0 = xf[:, 0]
    x1 = xf[:, 1]
    x2 = xf[:, 2]
    row = edges[0]
    col = edges[1]
    distances = _dist_sc(x0, x1, x2, row, col).reshape(E, 1)

    catf = categories.astype(jnp.float32).reshape(NF, 1)
    ch = charges.reshape(NF, 1)
    nm = node_mask.reshape(NF, 1)
    emb_pad = jnp.zeros((DIM, DIM), jnp.float32).at[:MAX_Z, 1:DIM].set(emb_table)
    parameters = _param_tc(catf, ch, nm, emb_pad, W_ml, b_ml.reshape(1, 2 * DIM))

    em = edge_mask.reshape(E, 1)
    return parameters, distances, edges, nm, em


# speedup vs baseline: 87.7392x; 87.7392x over previous
import functools

import jax
import jax.numpy as jnp
from jax import lax
from jax.experimental import pallas as pl
from jax.experimental.pallas import tpu as pltpu
from jax.experimental.pallas import tpu_sc as plsc

B, N_NODES, DIM, MAX_Z = 256, 64, 128, 100
NF = B * N_NODES
E = B * N_NODES * N_NODES

NW = 32
E_PER_W = E // NW
CHUNK = 8192
N_CHUNKS = E_PER_W // CHUNK
LANES = 16

_sc_mesh = plsc.VectorSubcoreMesh(core_axis_name="c", subcore_axis_name="s")


@functools.partial(
    pl.kernel,
    mesh=_sc_mesh,
    compiler_params=pltpu.CompilerParams(needs_layout_passes=False),
    out_type=jax.ShapeDtypeStruct((E,), jnp.float32),
    scratch_types=[
        pltpu.VMEM((NF,), jnp.float32),
        pltpu.VMEM((NF,), jnp.float32),
        pltpu.VMEM((NF,), jnp.float32),
        pltpu.VMEM((CHUNK,), jnp.int32),
        pltpu.VMEM((CHUNK,), jnp.int32),
        pltpu.VMEM((CHUNK,), jnp.float32),
    ],
)
def _dist_sc(x0_hbm, x1_hbm, x2_hbm, row_hbm, col_hbm, out_hbm,
             x0_v, x1_v, x2_v, row_v, col_v, out_v):
    wid = lax.axis_index("s") * 2 + lax.axis_index("c")
    base = wid * E_PER_W
    pltpu.sync_copy(x0_hbm, x0_v)
    pltpu.sync_copy(x1_hbm, x1_v)
    pltpu.sync_copy(x2_hbm, x2_v)

    def chunk_body(ci, carry):
        off = base + ci * CHUNK
        pltpu.sync_copy(row_hbm.at[pl.ds(off, CHUNK)], row_v)
        pltpu.sync_copy(col_hbm.at[pl.ds(off, CHUNK)], col_v)

        def step(i, c2):
            r = row_v[pl.ds(i * LANES, LANES)]
            c = col_v[pl.ds(i * LANES, LANES)]
            dx = plsc.load_gather(x0_v, [r]) - plsc.load_gather(x0_v, [c])
            dy = plsc.load_gather(x1_v, [r]) - plsc.load_gather(x1_v, [c])
            dz = plsc.load_gather(x2_v, [r]) - plsc.load_gather(x2_v, [c])
            out_v[pl.ds(i * LANES, LANES)] = dx * dx + dy * dy + dz * dz
            return c2

        lax.fori_loop(0, CHUNK // LANES, step, 0)
        pltpu.sync_copy(out_v, out_hbm.at[pl.ds(off, CHUNK)])
        return carry

    lax.fori_loop(0, N_CHUNKS, chunk_body, 0)


ROWS = 1024
GRID = NF // ROWS


def _param_body(cat_ref, ch_ref, nm_ref, emb_ref, w_ref, b_ref, out_ref):
    cat = cat_ref[...]
    z = lax.broadcasted_iota(jnp.int32, (1, DIM), 1).astype(jnp.float32)
    onehot = (cat == z).astype(jnp.float32)
    h = jnp.dot(onehot, emb_ref[...], preferred_element_type=jnp.float32)
    lane = lax.broadcasted_iota(jnp.int32, (ROWS, DIM), 1)
    h = jnp.where(lane == 0, ch_ref[...], h)
    h = h * nm_ref[...]
    out_ref[...] = (
        jnp.dot(h, w_ref[...], preferred_element_type=jnp.float32) + b_ref[...]
    )


_param_tc = pl.pallas_call(
    _param_body,
    grid=(GRID,),
    in_specs=[
        pl.BlockSpec((ROWS, 1), lambda i: (i, 0)),
        pl.BlockSpec((ROWS, 1), lambda i: (i, 0)),
        pl.BlockSpec((ROWS, 1), lambda i: (i, 0)),
        pl.BlockSpec((DIM, DIM), lambda i: (0, 0)),
        pl.BlockSpec((DIM, 2 * DIM), lambda i: (0, 0)),
        pl.BlockSpec((1, 2 * DIM), lambda i: (0, 0)),
    ],
    out_specs=pl.BlockSpec((ROWS, 2 * DIM), lambda i: (i, 0)),
    out_shape=jax.ShapeDtypeStruct((NF, 2 * DIM), jnp.float32),
)


def kernel(x, categories, charges, edges, node_mask, edge_mask,
           emb_table, W_ml, b_ml):
    xf = x.reshape(NF, 3)
    x0 = xf[:, 0]
    x1 = xf[:, 1]
    x2 = xf[:, 2]
    row = edges[0]
    col = edges[1]
    distances = _dist_sc(x0, x1, x2, row, col).reshape(E, 1)

    catf = categories.astype(jnp.float32).reshape(NF, 1)
    ch = charges.reshape(NF, 1)
    nm = node_mask.reshape(NF, 1)
    emb_pad = jnp.zeros((DIM, DIM), jnp.float32).at[:MAX_Z, 1:DIM].set(emb_table)
    parameters = _param_tc(catf, ch, nm, emb_pad, W_ml, b_ml.reshape(1, 2 * DIM))

    em = edge_mask.reshape(E, 1)
    return parameters, distances, edges, nm, em


# speedup vs baseline: 88.4066x; 1.0076x over previous
import functools

import jax
import jax.numpy as jnp
from jax import lax
from jax.experimental import pallas as pl
from jax.experimental.pallas import tpu as pltpu
from jax.experimental.pallas import tpu_sc as plsc

B, N_NODES, DIM, MAX_Z = 256, 64, 128, 100
NF = B * N_NODES
E = B * N_NODES * N_NODES

NW = 32
E_PER_W = E // NW
CHUNK = 8192
N_CHUNKS = E_PER_W // CHUNK
LANES = 16

_sc_mesh = plsc.VectorSubcoreMesh(core_axis_name="c", subcore_axis_name="s")


@functools.partial(
    pl.kernel,
    mesh=_sc_mesh,
    compiler_params=pltpu.CompilerParams(needs_layout_passes=False),
    out_type=jax.ShapeDtypeStruct((E,), jnp.float32),
    scratch_types=[
        pltpu.VMEM((NF,), jnp.float32),
        pltpu.VMEM((NF,), jnp.float32),
        pltpu.VMEM((NF,), jnp.float32),
        pltpu.VMEM((2, CHUNK), jnp.int32),
        pltpu.VMEM((2, CHUNK), jnp.int32),
        pltpu.VMEM((2, CHUNK), jnp.float32),
        pltpu.SemaphoreType.DMA,
        pltpu.SemaphoreType.DMA,
        pltpu.SemaphoreType.DMA,
        pltpu.SemaphoreType.DMA,
        pltpu.SemaphoreType.DMA,
    ],
)
def _dist_sc(x0_hbm, x1_hbm, x2_hbm, row_hbm, col_hbm, out_hbm,
             x0_v, x1_v, x2_v, row_v, col_v, out_v,
             sem_x, sem_in0, sem_in1, sem_out0, sem_out1):
    wid = lax.axis_index("s") * 2 + lax.axis_index("c")
    base = wid * E_PER_W
    sems_in = (sem_in0, sem_in1)
    sems_out = (sem_out0, sem_out1)

    hx = [pltpu.async_copy(s, d, sem_x)
          for s, d in ((x0_hbm, x0_v), (x1_hbm, x1_v), (x2_hbm, x2_v))]

    def issue_in(ci):
        b = ci % 2
        off = base + ci * CHUNK
        return (
            pltpu.async_copy(row_hbm.at[pl.ds(off, CHUNK)], row_v.at[b],
                             sems_in[b]),
            pltpu.async_copy(col_hbm.at[pl.ds(off, CHUNK)], col_v.at[b],
                             sems_in[b]),
        )

    in_h = {0: issue_in(0), 1: issue_in(1)}
    for h in hx:
        h.wait()

    out_h = {}
    for ci in range(N_CHUNKS):
        b = ci % 2
        for h in in_h.pop(ci):
            h.wait()
        if ci >= 2:
            out_h.pop(ci - 2).wait()

        @plsc.parallel_loop(0, CHUNK // LANES, 1, unroll=8)
        def _body(i):
            r = row_v[b, pl.ds(i * LANES, LANES)]
            c = col_v[b, pl.ds(i * LANES, LANES)]
            dx = plsc.load_gather(x0_v, [r]) - plsc.load_gather(x0_v, [c])
            dy = plsc.load_gather(x1_v, [r]) - plsc.load_gather(x1_v, [c])
            dz = plsc.load_gather(x2_v, [r]) - plsc.load_gather(x2_v, [c])
            out_v[b, pl.ds(i * LANES, LANES)] = dx * dx + dy * dy + dz * dz

        out_h[ci] = pltpu.async_copy(
            out_v.at[b], out_hbm.at[pl.ds(base + ci * CHUNK, CHUNK)],
            sems_out[b])
        if ci + 2 < N_CHUNKS:
            in_h[ci + 2] = issue_in(ci + 2)
    for h in out_h.values():
        h.wait()


ROWS = 1024
GRID = NF // ROWS


def _param_body(cat_ref, ch_ref, nm_ref, emb_ref, w_ref, b_ref, out_ref):
    cat = cat_ref[...]
    z = lax.broadcasted_iota(jnp.int32, (1, DIM), 1).astype(jnp.float32)
    onehot = (cat == z).astype(jnp.float32)
    h = jnp.dot(onehot, emb_ref[...], preferred_element_type=jnp.float32)
    lane = lax.broadcasted_iota(jnp.int32, (ROWS, DIM), 1)
    h = jnp.where(lane == 0, ch_ref[...], h)
    h = h * nm_ref[...]
    out_ref[...] = (
        jnp.dot(h, w_ref[...], preferred_element_type=jnp.float32) + b_ref[...]
    )


_param_tc = pl.pallas_call(
    _param_body,
    grid=(GRID,),
    in_specs=[
        pl.BlockSpec((ROWS, 1), lambda i: (i, 0)),
        pl.BlockSpec((ROWS, 1), lambda i: (i, 0)),
        pl.BlockSpec((ROWS, 1), lambda i: (i, 0)),
        pl.BlockSpec((DIM, DIM), lambda i: (0, 0)),
        pl.BlockSpec((DIM, 2 * DIM), lambda i: (0, 0)),
        pl.BlockSpec((1, 2 * DIM), lambda i: (0, 0)),
    ],
    out_specs=pl.BlockSpec((ROWS, 2 * DIM), lambda i: (i, 0)),
    out_shape=jax.ShapeDtypeStruct((NF, 2 * DIM), jnp.float32),
)


def kernel(x, categories, charges, edges, node_mask, edge_mask,
           emb_table, W_ml, b_ml):
    xf = x.reshape(NF, 3)
    x0 = xf[:, 0]
    x1 = xf[:, 1]
    x2 = xf[:, 2]
    row = edges[0]
    col = edges[1]
    distances = _dist_sc(x0, x1, x2, row, col).reshape(E, 1)

    catf = categories.astype(jnp.float32).reshape(NF, 1)
    ch = charges.reshape(NF, 1)
    nm = node_mask.reshape(NF, 1)
    emb_pad = jnp.zeros((DIM, DIM), jnp.float32).at[:MAX_Z, 1:DIM].set(emb_table)
    parameters = _param_tc(catf, ch, nm, emb_pad, W_ml, b_ml.reshape(1, 2 * DIM))

    em = edge_mask.reshape(E, 1)
    return parameters, distances, edges, nm, em
